# baseline (speedup 1.0000x reference)
import functools

import jax
import jax.numpy as jnp
from jax import lax
from jax.experimental import pallas as pl
from jax.experimental.pallas import tpu as pltpu
from jax.experimental.pallas import tpu_sc as plsc

B = 32
K = 1024
DIM = 2048
KNN = 8
N_CLASS = 16
INV_T = 20.0

NC = 2
NS = 16
L = 16

KTC = 512
KSC = K - KTC

CH = 16
NBUF = 3
NCHUNK = KSC // CH
STEPS = (NCHUNK - 1) // NBUF
UNROLL = 2
DGRP = DIM // (L * UNROLL)

BK = 128
BB = 8
MCH = KTC // L

BIG = 3.0e38

_mesh = plsc.VectorSubcoreMesh(core_axis_name="c", subcore_axis_name="s")
_sc_params = pltpu.CompilerParams(needs_layout_passes=False)


def _merge_sorted(top_d, top_l, sd, sl):
    sdr = jnp.flip(sd)
    slr = jnp.flip(sl)
    sel = top_d <= sdr
    md = jnp.where(sel, top_d, sdr)
    ml = jnp.where(sel, top_l, slr)
    rd, rl = plsc.sort_key_val(md, ml)
    return rd, rl


@functools.partial(
    pl.kernel,
    out_type=(
        jax.ShapeDtypeStruct((B, L), jnp.float32),
        jax.ShapeDtypeStruct((B, L), jnp.int32),
    ),
    mesh=_mesh,
    compiler_params=_sc_params,
    scratch_types=[
        pltpu.VMEM((DIM,), jnp.float32),
        pltpu.VMEM((KSC,), jnp.int32),
        pltpu.VMEM((CH, DIM), jnp.float32),
        pltpu.VMEM((CH, DIM), jnp.float32),
        pltpu.VMEM((CH, DIM), jnp.float32),
        pltpu.VMEM((L,), jnp.float32),
        pltpu.VMEM((L,), jnp.int32),
        pltpu.SemaphoreType.DMA,
        pltpu.SemaphoreType.DMA,
        pltpu.SemaphoreType.DMA,
    ],
)
def _sc_partial(logits_hbm, qa_hbm, ql_hbm, outd_hbm, outl_hbm,
                l_ref, lab_ref, buf0, buf1, buf2, tdv, tlv,
                sem0, sem1, sem2):
    b = lax.axis_index("s") * NC + lax.axis_index("c")
    bufs = (buf0, buf1, buf2)
    sems = (sem0, sem1, sem2)

    pltpu.sync_copy(logits_hbm.at[b], l_ref)
    pltpu.sync_copy(ql_hbm.at[b, pl.ds(KTC, KSC)], lab_ref)

    for i in range(NBUF):
        pltpu.async_copy(
            qa_hbm.at[b, pl.ds(KTC + i * CH, CH), :], bufs[i], sems[i])

    def chunk_dists(buf):
        def dim_body(j, accs):
            accs = list(accs)
            for u in range(UNROLL):
                base = (j * UNROLL + u) * L
                lvec = l_ref[pl.ds(base, L)]
                for a in range(CH):
                    d = buf[a, pl.ds(base, L)] - lvec
                    accs[a] = accs[a] + d * d
            return tuple(accs)

        z = jnp.zeros((L,), jnp.float32)
        accs = lax.fori_loop(0, DGRP, dim_body, (z,) * CH)
        lanes = lax.iota(jnp.int32, L)
        dvec = jnp.zeros((L,), jnp.float32)
        for a in range(CH):
            dvec = jnp.where(lanes == a, jnp.sum(accs[a]), dvec)
        return dvec

    def consume(k, i, top_d, top_l, refill):
        src = qa_hbm.at[b, pl.ds(KTC + k * CH, CH), :]
        pltpu.make_async_copy(src, bufs[i], sems[i]).wait()

        dvec = chunk_dists(bufs[i])
        lab16 = lab_ref[pl.ds(k * CH, L)]

        if refill:
            nk = k + NBUF

            @pl.when(nk < NCHUNK)
            def _():
                pltpu.async_copy(
                    qa_hbm.at[b, pl.ds(KTC + nk * CH, CH), :],
                    bufs[i], sems[i])

        sd, sl = plsc.sort_key_val(dvec, lab16)
        return _merge_sorted(top_d, top_l, sd, sl)

    def step(s, carry):
        top_d, top_l = carry
        for i in range(NBUF):
            top_d, top_l = consume(s * NBUF + i, i, top_d, top_l, refill=True)
        return top_d, top_l

    top_d = jnp.full((L,), BIG, jnp.float32)
    top_l = jnp.zeros((L,), jnp.int32)
    top_d, top_l = lax.fori_loop(0, STEPS, step, (top_d, top_l))
    for k in range(STEPS * NBUF, NCHUNK):
        top_d, top_l = consume(k, k % NBUF, top_d, top_l, refill=False)

    tdv[...] = top_d
    tlv[...] = top_l
    pltpu.sync_copy(tdv, outd_hbm.at[b])
    pltpu.sync_copy(tlv, outl_hbm.at[b])


def _tc_body(l_ref, qa_ref, o_ref):
    d = qa_ref[...] - l_ref[...][:, None, :]
    o_ref[...] = jnp.sum(d * d, axis=-1)


_tc_dists = pl.pallas_call(
    _tc_body,
    grid=(B // BB, KTC // BK),
    in_specs=[
        pl.BlockSpec((BB, DIM), lambda i, k: (i, 0)),
        pl.BlockSpec((BB, BK, DIM), lambda i, k: (i, k, 0)),
    ],
    out_specs=pl.BlockSpec((BB, BK), lambda i, k: (i, k)),
    out_shape=jax.ShapeDtypeStruct((B, KTC), jnp.float32),
)


@functools.partial(
    pl.kernel,
    out_type=jax.ShapeDtypeStruct((B, N_CLASS), jnp.float32),
    mesh=_mesh,
    compiler_params=_sc_params,
    scratch_types=[
        pltpu.VMEM((KTC,), jnp.float32),
        pltpu.VMEM((KTC,), jnp.int32),
        pltpu.VMEM((L,), jnp.float32),
        pltpu.VMEM((L,), jnp.int32),
        pltpu.VMEM((N_CLASS,), jnp.float32),
    ],
)
def _sc_merge(tcd_hbm, scd_hbm, scl_hbm, ql_hbm, out_hbm,
              td_ref, lab_ref, scd_ref, scl_ref, outv):
    b = lax.axis_index("s") * NC + lax.axis_index("c")

    pltpu.sync_copy(tcd_hbm.at[b], td_ref)
    pltpu.sync_copy(ql_hbm.at[b, pl.ds(0, KTC)], lab_ref)
    pltpu.sync_copy(scd_hbm.at[b], scd_ref)
    pltpu.sync_copy(scl_hbm.at[b], scl_ref)

    lanes = lax.iota(jnp.int32, L)

    def step(c, carry):
        top_d, top_i = carry
        dc = td_ref[pl.ds(c * L, L)]
        ic = lanes + c * L
        sd, si = plsc.sort_key_val(dc, ic)
        return _merge_sorted(top_d, top_i, sd, si)

    top_d = jnp.full((L,), BIG, jnp.float32)
    top_i = jnp.zeros((L,), jnp.int32)
    top_d, top_i = lax.fori_loop(0, MCH, step, (top_d, top_i))

    tc_lab = plsc.load_gather(lab_ref, [top_i])
    top_d, top_l = _merge_sorted(top_d, tc_lab, scd_ref[...], scl_ref[...])

    valid = lanes < KNN
    s = jnp.where(valid, -INV_T * top_d, -1e30)
    m = jnp.max(s)
    e = jnp.exp(s - m)
    tot = jnp.sum(e)
    w = e / tot

    acc = jnp.zeros((N_CLASS,), jnp.float32)
    for i in range(KNN):
        acc = acc + jnp.where(lanes == top_l[i], w[i], 0.0)
    outv[...] = acc
    pltpu.sync_copy(outv, out_hbm.at[b])


def kernel(logits, queue_anchor, queue_label):
    scd, scl = _sc_partial(logits, queue_anchor, queue_label)
    tcd = _tc_dists(logits, queue_anchor)
    return _sc_merge(tcd, scd, scl, queue_label)


# speedup vs baseline: 1.0610x; 1.0610x over previous
import functools

import jax
import jax.numpy as jnp
from jax import lax
from jax.experimental import pallas as pl
from jax.experimental.pallas import tpu as pltpu
from jax.experimental.pallas import tpu_sc as plsc

B = 32
K = 1024
DIM = 2048
KNN = 8
N_CLASS = 16
INV_T = 20.0

NC = 2
NS = 16
L = 16

KTC = 768
KSC = K - KTC

CH = 16
NBUF = 3
NCHUNK = KSC // CH
STEPS = (NCHUNK - 1) // NBUF
UNROLL = 2
DGRP = DIM // (L * UNROLL)

BK = 128
BB = 8
MCH = KTC // L

BIG = 3.0e38

_mesh = plsc.VectorSubcoreMesh(core_axis_name="c", subcore_axis_name="s")
_sc_params = pltpu.CompilerParams(needs_layout_passes=False)


def _merge_sorted(top_d, top_l, sd, sl):
    sdr = jnp.flip(sd)
    slr = jnp.flip(sl)
    sel = top_d <= sdr
    md = jnp.where(sel, top_d, sdr)
    ml = jnp.where(sel, top_l, slr)
    rd, rl = plsc.sort_key_val(md, ml)
    return rd, rl


@functools.partial(
    pl.kernel,
    out_type=(
        jax.ShapeDtypeStruct((B, L), jnp.float32),
        jax.ShapeDtypeStruct((B, L), jnp.int32),
    ),
    mesh=_mesh,
    compiler_params=_sc_params,
    scratch_types=[
        pltpu.VMEM((DIM,), jnp.float32),
        pltpu.VMEM((KSC,), jnp.int32),
        pltpu.VMEM((CH, DIM), jnp.float32),
        pltpu.VMEM((CH, DIM), jnp.float32),
        pltpu.VMEM((CH, DIM), jnp.float32),
        pltpu.VMEM((L,), jnp.float32),
        pltpu.VMEM((L,), jnp.int32),
        pltpu.SemaphoreType.DMA,
        pltpu.SemaphoreType.DMA,
        pltpu.SemaphoreType.DMA,
    ],
)
def _sc_partial(logits_hbm, qa_hbm, ql_hbm, outd_hbm, outl_hbm,
                l_ref, lab_ref, buf0, buf1, buf2, tdv, tlv,
                sem0, sem1, sem2):
    b = lax.axis_index("s") * NC + lax.axis_index("c")
    bufs = (buf0, buf1, buf2)
    sems = (sem0, sem1, sem2)

    pltpu.sync_copy(logits_hbm.at[b], l_ref)
    pltpu.sync_copy(ql_hbm.at[b, pl.ds(KTC, KSC)], lab_ref)

    for i in range(NBUF):
        pltpu.async_copy(
            qa_hbm.at[b, pl.ds(KTC + i * CH, CH), :], bufs[i], sems[i])

    def chunk_dists(buf):
        def dim_body(j, accs):
            accs = list(accs)
            for u in range(UNROLL):
                base = (j * UNROLL + u) * L
                lvec = l_ref[pl.ds(base, L)]
                for a in range(CH):
                    d = buf[a, pl.ds(base, L)] - lvec
                    accs[a] = accs[a] + d * d
            return tuple(accs)

        z = jnp.zeros((L,), jnp.float32)
        accs = lax.fori_loop(0, DGRP, dim_body, (z,) * CH)
        lanes = lax.iota(jnp.int32, L)
        dvec = jnp.zeros((L,), jnp.float32)
        for a in range(CH):
            dvec = jnp.where(lanes == a, jnp.sum(accs[a]), dvec)
        return dvec

    def consume(k, i, top_d, top_l, refill):
        src = qa_hbm.at[b, pl.ds(KTC + k * CH, CH), :]
        pltpu.make_async_copy(src, bufs[i], sems[i]).wait()

        dvec = chunk_dists(bufs[i])
        lab16 = lab_ref[pl.ds(k * CH, L)]

        if refill:
            nk = k + NBUF

            @pl.when(nk < NCHUNK)
            def _():
                pltpu.async_copy(
                    qa_hbm.at[b, pl.ds(KTC + nk * CH, CH), :],
                    bufs[i], sems[i])

        sd, sl = plsc.sort_key_val(dvec, lab16)
        return _merge_sorted(top_d, top_l, sd, sl)

    def step(s, carry):
        top_d, top_l = carry
        for i in range(NBUF):
            top_d, top_l = consume(s * NBUF + i, i, top_d, top_l, refill=True)
        return top_d, top_l

    top_d = jnp.full((L,), BIG, jnp.float32)
    top_l = jnp.zeros((L,), jnp.int32)
    top_d, top_l = lax.fori_loop(0, STEPS, step, (top_d, top_l))
    for k in range(STEPS * NBUF, NCHUNK):
        top_d, top_l = consume(k, k % NBUF, top_d, top_l, refill=False)

    tdv[...] = top_d
    tlv[...] = top_l
    pltpu.sync_copy(tdv, outd_hbm.at[b])
    pltpu.sync_copy(tlv, outl_hbm.at[b])


def _tc_body(l_ref, qa_ref, o_ref):
    d = qa_ref[...] - l_ref[...][:, None, :]
    o_ref[...] = jnp.sum(d * d, axis=-1)


_tc_dists = pl.pallas_call(
    _tc_body,
    grid=(B // BB, KTC // BK),
    in_specs=[
        pl.BlockSpec((BB, DIM), lambda i, k: (i, 0)),
        pl.BlockSpec((BB, BK, DIM), lambda i, k: (i, k, 0)),
    ],
    out_specs=pl.BlockSpec((BB, BK), lambda i, k: (i, k)),
    out_shape=jax.ShapeDtypeStruct((B, KTC), jnp.float32),
)


@functools.partial(
    pl.kernel,
    out_type=jax.ShapeDtypeStruct((B, N_CLASS), jnp.float32),
    mesh=_mesh,
    compiler_params=_sc_params,
    scratch_types=[
        pltpu.VMEM((KTC,), jnp.float32),
        pltpu.VMEM((KTC,), jnp.int32),
        pltpu.VMEM((L,), jnp.float32),
        pltpu.VMEM((L,), jnp.int32),
        pltpu.VMEM((N_CLASS,), jnp.float32),
    ],
)
def _sc_merge(tcd_hbm, scd_hbm, scl_hbm, ql_hbm, out_hbm,
              td_ref, lab_ref, scd_ref, scl_ref, outv):
    b = lax.axis_index("s") * NC + lax.axis_index("c")

    pltpu.sync_copy(tcd_hbm.at[b], td_ref)
    pltpu.sync_copy(ql_hbm.at[b, pl.ds(0, KTC)], lab_ref)
    pltpu.sync_copy(scd_hbm.at[b], scd_ref)
    pltpu.sync_copy(scl_hbm.at[b], scl_ref)

    lanes = lax.iota(jnp.int32, L)

    def step(c, carry):
        top_d, top_i = carry
        dc = td_ref[pl.ds(c * L, L)]
        ic = lanes + c * L
        sd, si = plsc.sort_key_val(dc, ic)
        return _merge_sorted(top_d, top_i, sd, si)

    top_d = jnp.full((L,), BIG, jnp.float32)
    top_i = jnp.zeros((L,), jnp.int32)
    top_d, top_i = lax.fori_loop(0, MCH, step, (top_d, top_i))

    tc_lab = plsc.load_gather(lab_ref, [top_i])
    top_d, top_l = _merge_sorted(top_d, tc_lab, scd_ref[...], scl_ref[...])

    valid = lanes < KNN
    s = jnp.where(valid, -INV_T * top_d, -1e30)
    m = jnp.max(s)
    e = jnp.exp(s - m)
    tot = jnp.sum(e)
    w = e / tot

    acc = jnp.zeros((N_CLASS,), jnp.float32)
    for i in range(KNN):
        acc = acc + jnp.where(lanes == top_l[i], w[i], 0.0)
    outv[...] = acc
    pltpu.sync_copy(outv, out_hbm.at[b])


def kernel(logits, queue_anchor, queue_label):
    scd, scl = _sc_partial(logits, queue_anchor, queue_label)
    tcd = _tc_dists(logits, queue_anchor)
    return _sc_merge(tcd, scd, scl, queue_label)


# speedup vs baseline: 1.0722x; 1.0106x over previous
import functools

import jax
import jax.numpy as jnp
from jax import lax
from jax.experimental import pallas as pl
from jax.experimental.pallas import tpu as pltpu
from jax.experimental.pallas import tpu_sc as plsc

B = 32
K = 1024
DIM = 2048
KNN = 8
N_CLASS = 16
INV_T = 20.0

NC = 2
NS = 16
L = 16

KTC = 896
KSC = K - KTC

CH = 16
NBUF = 3
NCHUNK = KSC // CH
STEPS = (NCHUNK - 1) // NBUF
UNROLL = 2
DGRP = DIM // (L * UNROLL)

BK = 128
BB = 8
MCH = KTC // L

BIG = 3.0e38

_mesh = plsc.VectorSubcoreMesh(core_axis_name="c", subcore_axis_name="s")
_sc_params = pltpu.CompilerParams(needs_layout_passes=False)


def _merge_sorted(top_d, top_l, sd, sl):
    sdr = jnp.flip(sd)
    slr = jnp.flip(sl)
    sel = top_d <= sdr
    md = jnp.where(sel, top_d, sdr)
    ml = jnp.where(sel, top_l, slr)
    rd, rl = plsc.sort_key_val(md, ml)
    return rd, rl


@functools.partial(
    pl.kernel,
    out_type=(
        jax.ShapeDtypeStruct((B, L), jnp.float32),
        jax.ShapeDtypeStruct((B, L), jnp.int32),
    ),
    mesh=_mesh,
    compiler_params=_sc_params,
    scratch_types=[
        pltpu.VMEM((DIM,), jnp.float32),
        pltpu.VMEM((KSC,), jnp.int32),
        pltpu.VMEM((CH, DIM), jnp.float32),
        pltpu.VMEM((CH, DIM), jnp.float32),
        pltpu.VMEM((CH, DIM), jnp.float32),
        pltpu.VMEM((L,), jnp.float32),
        pltpu.VMEM((L,), jnp.int32),
        pltpu.SemaphoreType.DMA,
        pltpu.SemaphoreType.DMA,
        pltpu.SemaphoreType.DMA,
    ],
)
def _sc_partial(logits_hbm, qa_hbm, ql_hbm, outd_hbm, outl_hbm,
                l_ref, lab_ref, buf0, buf1, buf2, tdv, tlv,
                sem0, sem1, sem2):
    b = lax.axis_index("s") * NC + lax.axis_index("c")
    bufs = (buf0, buf1, buf2)
    sems = (sem0, sem1, sem2)

    pltpu.sync_copy(logits_hbm.at[b], l_ref)
    pltpu.sync_copy(ql_hbm.at[b, pl.ds(KTC, KSC)], lab_ref)

    for i in range(NBUF):
        pltpu.async_copy(
            qa_hbm.at[b, pl.ds(KTC + i * CH, CH), :], bufs[i], sems[i])

    def chunk_dists(buf):
        def dim_body(j, accs):
            accs = list(accs)
            for u in range(UNROLL):
                base = (j * UNROLL + u) * L
                lvec = l_ref[pl.ds(base, L)]
                for a in range(CH):
                    d = buf[a, pl.ds(base, L)] - lvec
                    accs[a] = accs[a] + d * d
            return tuple(accs)

        z = jnp.zeros((L,), jnp.float32)
        accs = lax.fori_loop(0, DGRP, dim_body, (z,) * CH)
        lanes = lax.iota(jnp.int32, L)
        dvec = jnp.zeros((L,), jnp.float32)
        for a in range(CH):
            dvec = jnp.where(lanes == a, jnp.sum(accs[a]), dvec)
        return dvec

    def consume(k, i, top_d, top_l, refill):
        src = qa_hbm.at[b, pl.ds(KTC + k * CH, CH), :]
        pltpu.make_async_copy(src, bufs[i], sems[i]).wait()

        dvec = chunk_dists(bufs[i])
        lab16 = lab_ref[pl.ds(k * CH, L)]

        if refill:
            nk = k + NBUF

            @pl.when(nk < NCHUNK)
            def _():
                pltpu.async_copy(
                    qa_hbm.at[b, pl.ds(KTC + nk * CH, CH), :],
                    bufs[i], sems[i])

        sd, sl = plsc.sort_key_val(dvec, lab16)
        return _merge_sorted(top_d, top_l, sd, sl)

    def step(s, carry):
        top_d, top_l = carry
        for i in range(NBUF):
            top_d, top_l = consume(s * NBUF + i, i, top_d, top_l, refill=True)
        return top_d, top_l

    top_d = jnp.full((L,), BIG, jnp.float32)
    top_l = jnp.zeros((L,), jnp.int32)
    top_d, top_l = lax.fori_loop(0, STEPS, step, (top_d, top_l))
    for k in range(STEPS * NBUF, NCHUNK):
        top_d, top_l = consume(k, k % NBUF, top_d, top_l, refill=False)

    tdv[...] = top_d
    tlv[...] = top_l
    pltpu.sync_copy(tdv, outd_hbm.at[b])
    pltpu.sync_copy(tlv, outl_hbm.at[b])


def _tc_body(l_ref, qa_ref, o_ref):
    d = qa_ref[...] - l_ref[...][:, None, :]
    o_ref[...] = jnp.sum(d * d, axis=-1)


_tc_dists = pl.pallas_call(
    _tc_body,
    grid=(B // BB, KTC // BK),
    in_specs=[
        pl.BlockSpec((BB, DIM), lambda i, k: (i, 0)),
        pl.BlockSpec((BB, BK, DIM), lambda i, k: (i, k, 0)),
    ],
    out_specs=pl.BlockSpec((BB, BK), lambda i, k: (i, k)),
    out_shape=jax.ShapeDtypeStruct((B, KTC), jnp.float32),
)


@functools.partial(
    pl.kernel,
    out_type=jax.ShapeDtypeStruct((B, N_CLASS), jnp.float32),
    mesh=_mesh,
    compiler_params=_sc_params,
    scratch_types=[
        pltpu.VMEM((KTC,), jnp.float32),
        pltpu.VMEM((KTC,), jnp.int32),
        pltpu.VMEM((L,), jnp.float32),
        pltpu.VMEM((L,), jnp.int32),
        pltpu.VMEM((N_CLASS,), jnp.float32),
    ],
)
def _sc_merge(tcd_hbm, scd_hbm, scl_hbm, ql_hbm, out_hbm,
              td_ref, lab_ref, scd_ref, scl_ref, outv):
    b = lax.axis_index("s") * NC + lax.axis_index("c")

    pltpu.sync_copy(tcd_hbm.at[b], td_ref)
    pltpu.sync_copy(ql_hbm.at[b, pl.ds(0, KTC)], lab_ref)
    pltpu.sync_copy(scd_hbm.at[b], scd_ref)
    pltpu.sync_copy(scl_hbm.at[b], scl_ref)

    lanes = lax.iota(jnp.int32, L)

    def step(c, carry):
        top_d, top_i = carry
        dc = td_ref[pl.ds(c * L, L)]
        ic = lanes + c * L
        sd, si = plsc.sort_key_val(dc, ic)
        return _merge_sorted(top_d, top_i, sd, si)

    top_d = jnp.full((L,), BIG, jnp.float32)
    top_i = jnp.zeros((L,), jnp.int32)
    top_d, top_i = lax.fori_loop(0, MCH, step, (top_d, top_i))

    tc_lab = plsc.load_gather(lab_ref, [top_i])
    top_d, top_l = _merge_sorted(top_d, tc_lab, scd_ref[...], scl_ref[...])

    valid = lanes < KNN
    s = jnp.where(valid, -INV_T * top_d, -1e30)
    m = jnp.max(s)
    e = jnp.exp(s - m)
    tot = jnp.sum(e)
    w = e / tot

    acc = jnp.zeros((N_CLASS,), jnp.float32)
    for i in range(KNN):
        acc = acc + jnp.where(lanes == top_l[i], w[i], 0.0)
    outv[...] = acc
    pltpu.sync_copy(outv, out_hbm.at[b])


def kernel(logits, queue_anchor, queue_label):
    scd, scl = _sc_partial(logits, queue_anchor, queue_label)
    tcd = _tc_dists(logits, queue_anchor)
    return _sc_merge(tcd, scd, scl, queue_label)


# speedup vs baseline: 1.0769x; 1.0044x over previous
import functools

import jax
import jax.numpy as jnp
from jax import lax
from jax.experimental import pallas as pl
from jax.experimental.pallas import tpu as pltpu
from jax.experimental.pallas import tpu_sc as plsc

B = 32
K = 1024
DIM = 2048
KNN = 8
N_CLASS = 16
INV_T = 20.0

NC = 2
NS = 16
L = 16

KTC = 896
KSC = K - KTC

CH = 16
NBUF = 3
NCHUNK = KSC // CH
STEPS = (NCHUNK - 1) // NBUF
UNROLL = 2
DGRP = DIM // (L * UNROLL)

BK = 128
BB = 8
MCH = KTC // L

BIG = 3.0e38

_mesh = plsc.VectorSubcoreMesh(core_axis_name="c", subcore_axis_name="s")
_sc_params = pltpu.CompilerParams(needs_layout_passes=False)


def _merge_sorted(top_d, top_l, sd, sl):
    sdr = jnp.flip(sd)
    slr = jnp.flip(sl)
    sel = top_d <= sdr
    md = jnp.where(sel, top_d, sdr)
    ml = jnp.where(sel, top_l, slr)
    rd, rl = plsc.sort_key_val(md, ml)
    return rd, rl


@functools.partial(
    pl.kernel,
    out_type=(
        jax.ShapeDtypeStruct((B, L), jnp.float32),
        jax.ShapeDtypeStruct((B, L), jnp.int32),
    ),
    mesh=_mesh,
    compiler_params=_sc_params,
    scratch_types=[
        pltpu.VMEM((DIM,), jnp.float32),
        pltpu.VMEM((KSC,), jnp.int32),
        pltpu.VMEM((CH, DIM), jnp.float32),
        pltpu.VMEM((CH, DIM), jnp.float32),
        pltpu.VMEM((CH, DIM), jnp.float32),
        pltpu.VMEM((L,), jnp.float32),
        pltpu.VMEM((L,), jnp.int32),
        pltpu.SemaphoreType.DMA,
        pltpu.SemaphoreType.DMA,
        pltpu.SemaphoreType.DMA,
    ],
)
def _sc_partial(logits_hbm, qa_hbm, ql_hbm, outd_hbm, outl_hbm,
                l_ref, lab_ref, buf0, buf1, buf2, tdv, tlv,
                sem0, sem1, sem2):
    b = lax.axis_index("s") * NC + lax.axis_index("c")
    bufs = (buf0, buf1, buf2)
    sems = (sem0, sem1, sem2)

    pltpu.sync_copy(logits_hbm.at[b], l_ref)
    pltpu.sync_copy(ql_hbm.at[b, pl.ds(KTC, KSC)], lab_ref)

    for i in range(NBUF):
        pltpu.async_copy(
            qa_hbm.at[b, pl.ds(KTC + i * CH, CH), :], bufs[i], sems[i])

    def chunk_dists(buf):
        def dim_body(j, accs):
            accs = list(accs)
            for u in range(UNROLL):
                base = (j * UNROLL + u) * L
                lvec = l_ref[pl.ds(base, L)]
                for a in range(CH):
                    d = buf[a, pl.ds(base, L)] - lvec
                    accs[a] = accs[a] + d * d
            return tuple(accs)

        z = jnp.zeros((L,), jnp.float32)
        accs = lax.fori_loop(0, DGRP, dim_body, (z,) * CH)
        lanes = lax.iota(jnp.int32, L)
        dvec = jnp.zeros((L,), jnp.float32)
        for a in range(CH):
            dvec = jnp.where(lanes == a, jnp.sum(accs[a]), dvec)
        return dvec

    def consume(k, i, top_d, top_l, refill):
        src = qa_hbm.at[b, pl.ds(KTC + k * CH, CH), :]
        pltpu.make_async_copy(src, bufs[i], sems[i]).wait()

        dvec = chunk_dists(bufs[i])
        lab16 = lab_ref[pl.ds(k * CH, L)]

        if refill:
            nk = k + NBUF

            @pl.when(nk < NCHUNK)
            def _():
                pltpu.async_copy(
                    qa_hbm.at[b, pl.ds(KTC + nk * CH, CH), :],
                    bufs[i], sems[i])

        sd, sl = plsc.sort_key_val(dvec, lab16)
        return _merge_sorted(top_d, top_l, sd, sl)

    def step(s, carry):
        top_d, top_l = carry
        for i in range(NBUF):
            top_d, top_l = consume(s * NBUF + i, i, top_d, top_l, refill=True)
        return top_d, top_l

    top_d = jnp.full((L,), BIG, jnp.float32)
    top_l = jnp.zeros((L,), jnp.int32)
    top_d, top_l = lax.fori_loop(0, STEPS, step, (top_d, top_l))
    for k in range(STEPS * NBUF, NCHUNK):
        top_d, top_l = consume(k, k % NBUF, top_d, top_l, refill=False)

    tdv[...] = top_d
    tlv[...] = top_l
    pltpu.sync_copy(tdv, outd_hbm.at[b])
    pltpu.sync_copy(tlv, outl_hbm.at[b])


def _tc_body(l_ref, qa_ref, o_ref):
    d = qa_ref[...] - l_ref[...][:, None, :]
    o_ref[...] = jnp.sum(d * d, axis=-1)


_tc_dists = pl.pallas_call(
    _tc_body,
    grid=(B // BB, KTC // BK),
    in_specs=[
        pl.BlockSpec((BB, DIM), lambda i, k: (i, 0)),
        pl.BlockSpec((BB, BK, DIM), lambda i, k: (i, k, 0)),
    ],
    out_specs=pl.BlockSpec((BB, BK), lambda i, k: (i, k)),
    out_shape=jax.ShapeDtypeStruct((B, KTC), jnp.float32),
)


def _tc_finish_body(tcd_ref, lab_ref, scd_ref, scl_ref, o_ref):
    KA = KTC + L
    d_all = jnp.concatenate([tcd_ref[...], scd_ref[...]], axis=1)
    lab_all = jnp.concatenate([lab_ref[...], scl_ref[...]], axis=1)
    kio = jax.lax.broadcasted_iota(jnp.int32, (B, KA), 1)

    cur = d_all
    vals = []
    labs = []
    for _ in range(KNN):
        v = jnp.min(cur, axis=1, keepdims=True)
        cand = jnp.where(cur == v, kio, KA)
        idx = jnp.min(cand, axis=1, keepdims=True)
        hit = kio == idx
        labs.append(jnp.sum(jnp.where(hit, lab_all, 0), axis=1, keepdims=True))
        vals.append(v)
        cur = jnp.where(hit, BIG, cur)

    s = -INV_T * jnp.concatenate(vals, axis=1)
    m = jnp.max(s, axis=1, keepdims=True)
    e = jnp.exp(s - m)
    w = e / jnp.sum(e, axis=1, keepdims=True)

    cio = jax.lax.broadcasted_iota(jnp.int32, (B, N_CLASS), 1)
    acc = jnp.zeros((B, N_CLASS), jnp.float32)
    for r in range(KNN):
        acc = acc + w[:, r:r + 1] * (cio == labs[r]).astype(jnp.float32)
    o_ref[...] = acc


_tc_finish = pl.pallas_call(
    _tc_finish_body,
    out_shape=jax.ShapeDtypeStruct((B, N_CLASS), jnp.float32),
)


def kernel(logits, queue_anchor, queue_label):
    scd, scl = _sc_partial(logits, queue_anchor, queue_label)
    tcd = _tc_dists(logits, queue_anchor)
    return _tc_finish(tcd, queue_label[:, :KTC], scd, scl)


# speedup vs baseline: 1.0782x; 1.0012x over previous
import functools

import jax
import jax.numpy as jnp
from jax import lax
from jax.experimental import pallas as pl
from jax.experimental.pallas import tpu as pltpu
from jax.experimental.pallas import tpu_sc as plsc

B = 32
K = 1024
DIM = 2048
KNN = 8
N_CLASS = 16
INV_T = 20.0

NC = 2
NS = 16
L = 16

KTC = 896
KSC = K - KTC

CH = 16
NBUF = 2
NCHUNK = KSC // CH
STEPS = NCHUNK // NBUF
UNROLL = 2
DGRP = DIM // (L * UNROLL)

BK = 128
BB = 8

BIG = 3.0e38

_mesh = plsc.VectorSubcoreMesh(core_axis_name="c", subcore_axis_name="s")
_sc_params = pltpu.CompilerParams(needs_layout_passes=False)


def _merge_sorted(top_d, top_l, sd, sl):
    sdr = jnp.flip(sd)
    slr = jnp.flip(sl)
    sel = top_d <= sdr
    md = jnp.where(sel, top_d, sdr)
    ml = jnp.where(sel, top_l, slr)
    rd, rl = plsc.sort_key_val(md, ml)
    return rd, rl


@functools.partial(
    pl.kernel,
    out_type=(
        jax.ShapeDtypeStruct((B, L), jnp.float32),
        jax.ShapeDtypeStruct((B, L), jnp.int32),
    ),
    mesh=_mesh,
    compiler_params=_sc_params,
    scratch_types=[
        pltpu.VMEM((DIM,), jnp.float32),
        pltpu.VMEM((KSC,), jnp.int32),
        pltpu.VMEM((CH, DIM), jnp.float32),
        pltpu.VMEM((CH, DIM), jnp.float32),
        pltpu.VMEM((L,), jnp.float32),
        pltpu.VMEM((L,), jnp.int32),
        pltpu.SemaphoreType.DMA,
        pltpu.SemaphoreType.DMA,
    ],
)
def _sc_partial(logits_hbm, qa_hbm, ql_hbm, outd_hbm, outl_hbm,
                l_ref, lab_ref, buf0, buf1, tdv, tlv,
                sem0, sem1):
    b = lax.axis_index("s") * NC + lax.axis_index("c")
    bufs = (buf0, buf1)
    sems = (sem0, sem1)

    pltpu.sync_copy(logits_hbm.at[b], l_ref)
    pltpu.sync_copy(ql_hbm.at[b, pl.ds(KTC, KSC)], lab_ref)

    for i in range(NBUF):
        pltpu.async_copy(
            qa_hbm.at[b, pl.ds(KTC + i * CH, CH), :], bufs[i], sems[i])

    def chunk_dists(buf):
        def dim_body(j, accs):
            accs = list(accs)
            for u in range(UNROLL):
                base = (j * UNROLL + u) * L
                lvec = l_ref[pl.ds(base, L)]
                for a in range(CH):
                    d = buf[a, pl.ds(base, L)] - lvec
                    accs[a] = accs[a] + d * d
            return tuple(accs)

        z = jnp.zeros((L,), jnp.float32)
        accs = lax.fori_loop(0, DGRP, dim_body, (z,) * CH)
        lanes = lax.iota(jnp.int32, L)
        dvec = jnp.zeros((L,), jnp.float32)
        for a in range(CH):
            dvec = jnp.where(lanes == a, jnp.sum(accs[a]), dvec)
        return dvec

    def consume(k, i, top_d, top_l, refill):
        src = qa_hbm.at[b, pl.ds(KTC + k * CH, CH), :]
        pltpu.make_async_copy(src, bufs[i], sems[i]).wait()

        dvec = chunk_dists(bufs[i])
        lab16 = lab_ref[pl.ds(k * CH, L)]

        if refill:
            nk = k + NBUF

            @pl.when(nk < NCHUNK)
            def _():
                pltpu.async_copy(
                    qa_hbm.at[b, pl.ds(KTC + nk * CH, CH), :],
                    bufs[i], sems[i])

        sd, sl = plsc.sort_key_val(dvec, lab16)
        return _merge_sorted(top_d, top_l, sd, sl)

    def step(s, carry):
        top_d, top_l = carry
        for i in range(NBUF):
            top_d, top_l = consume(s * NBUF + i, i, top_d, top_l, refill=True)
        return top_d, top_l

    top_d = jnp.full((L,), BIG, jnp.float32)
    top_l = jnp.zeros((L,), jnp.int32)
    top_d, top_l = lax.fori_loop(0, STEPS, step, (top_d, top_l))
    for k in range(STEPS * NBUF, NCHUNK):
        top_d, top_l = consume(k, k % NBUF, top_d, top_l, refill=False)

    tdv[...] = top_d
    tlv[...] = top_l
    pltpu.sync_copy(tdv, outd_hbm.at[b])
    pltpu.sync_copy(tlv, outl_hbm.at[b])


def _tc_body(l_ref, qa_ref, o_ref):
    d = qa_ref[...] - l_ref[...][:, None, :]
    o_ref[...] = jnp.sum(d * d, axis=-1)


_tc_dists = pl.pallas_call(
    _tc_body,
    grid=(B // BB, KTC // BK),
    in_specs=[
        pl.BlockSpec((BB, DIM), lambda i, k: (i, 0)),
        pl.BlockSpec((BB, BK, DIM), lambda i, k: (i, k, 0)),
    ],
    out_specs=pl.BlockSpec((BB, BK), lambda i, k: (i, k)),
    out_shape=jax.ShapeDtypeStruct((B, KTC), jnp.float32),
)


def _tc_finish_body(tcd_ref, lab_ref, scd_ref, scl_ref, o_ref):
    KA = KTC + L
    d_all = jnp.concatenate([tcd_ref[...], scd_ref[...]], axis=1)
    lab_all = jnp.concatenate([lab_ref[...][:, :KTC], scl_ref[...]], axis=1)
    kio = jax.lax.broadcasted_iota(jnp.int32, (B, KA), 1)

    cur = d_all
    vals = []
    labs = []
    for _ in range(KNN):
        v = jnp.min(cur, axis=1, keepdims=True)
        cand = jnp.where(cur == v, kio, KA)
        idx = jnp.min(cand, axis=1, keepdims=True)
        hit = kio == idx
        labs.append(jnp.sum(jnp.where(hit, lab_all, 0), axis=1, keepdims=True))
        vals.append(v)
        cur = jnp.where(hit, BIG, cur)

    s = -INV_T * jnp.concatenate(vals, axis=1)
    m = jnp.max(s, axis=1, keepdims=True)
    e = jnp.exp(s - m)
    w = e / jnp.sum(e, axis=1, keepdims=True)

    cio = jax.lax.broadcasted_iota(jnp.int32, (B, N_CLASS), 1)
    acc = jnp.zeros((B, N_CLASS), jnp.float32)
    for r in range(KNN):
        acc = acc + w[:, r:r + 1] * (cio == labs[r]).astype(jnp.float32)
    o_ref[...] = acc


_tc_finish = pl.pallas_call(
    _tc_finish_body,
    out_shape=jax.ShapeDtypeStruct((B, N_CLASS), jnp.float32),
)


def kernel(logits, queue_anchor, queue_label):
    scd, scl = _sc_partial(logits, queue_anchor, queue_label)
    tcd = _tc_dists(logits, queue_anchor)
    return _tc_finish(tcd, queue_label, scd, scl)


# speedup vs baseline: 1.0825x; 1.0040x over previous
import functools

import jax
import jax.numpy as jnp
from jax import lax
from jax.experimental import pallas as pl
from jax.experimental.pallas import tpu as pltpu
from jax.experimental.pallas import tpu_sc as plsc

B = 32
K = 1024
DIM = 2048
KNN = 8
N_CLASS = 16
INV_T = 20.0

NC = 2
NS = 16
L = 16

KTC = 896
KSC = K - KTC

CH = 16
NBUF = 2
NCHUNK = KSC // CH
STEPS = NCHUNK // NBUF
UNROLL = 2
DGRP = DIM // (L * UNROLL)

BK = 128
BB = 16

BIG = 3.0e38

_mesh = plsc.VectorSubcoreMesh(core_axis_name="c", subcore_axis_name="s")
_sc_params = pltpu.CompilerParams(needs_layout_passes=False)


def _merge_sorted(top_d, top_l, sd, sl):
    sdr = jnp.flip(sd)
    slr = jnp.flip(sl)
    sel = top_d <= sdr
    md = jnp.where(sel, top_d, sdr)
    ml = jnp.where(sel, top_l, slr)
    rd, rl = plsc.sort_key_val(md, ml)
    return rd, rl


@functools.partial(
    pl.kernel,
    out_type=(
        jax.ShapeDtypeStruct((B, L), jnp.float32),
        jax.ShapeDtypeStruct((B, L), jnp.int32),
    ),
    mesh=_mesh,
    compiler_params=_sc_params,
    scratch_types=[
        pltpu.VMEM((DIM,), jnp.float32),
        pltpu.VMEM((KSC,), jnp.int32),
        pltpu.VMEM((CH, DIM), jnp.float32),
        pltpu.VMEM((CH, DIM), jnp.float32),
        pltpu.VMEM((L,), jnp.float32),
        pltpu.VMEM((L,), jnp.int32),
        pltpu.SemaphoreType.DMA,
        pltpu.SemaphoreType.DMA,
    ],
)
def _sc_partial(logits_hbm, qa_hbm, ql_hbm, outd_hbm, outl_hbm,
                l_ref, lab_ref, buf0, buf1, tdv, tlv,
                sem0, sem1):
    b = lax.axis_index("s") * NC + lax.axis_index("c")
    bufs = (buf0, buf1)
    sems = (sem0, sem1)

    pltpu.sync_copy(logits_hbm.at[b], l_ref)
    pltpu.sync_copy(ql_hbm.at[b, pl.ds(KTC, KSC)], lab_ref)

    for i in range(NBUF):
        pltpu.async_copy(
            qa_hbm.at[b, pl.ds(KTC + i * CH, CH), :], bufs[i], sems[i])

    def chunk_dists(buf):
        def dim_body(j, accs):
            accs = list(accs)
            for u in range(UNROLL):
                base = (j * UNROLL + u) * L
                lvec = l_ref[pl.ds(base, L)]
                for a in range(CH):
                    d = buf[a, pl.ds(base, L)] - lvec
                    accs[a] = accs[a] + d * d
            return tuple(accs)

        z = jnp.zeros((L,), jnp.float32)
        accs = lax.fori_loop(0, DGRP, dim_body, (z,) * CH)
        lanes = lax.iota(jnp.int32, L)
        dvec = jnp.zeros((L,), jnp.float32)
        for a in range(CH):
            dvec = jnp.where(lanes == a, jnp.sum(accs[a]), dvec)
        return dvec

    def consume(k, i, top_d, top_l, refill):
        src = qa_hbm.at[b, pl.ds(KTC + k * CH, CH), :]
        pltpu.make_async_copy(src, bufs[i], sems[i]).wait()

        dvec = chunk_dists(bufs[i])
        lab16 = lab_ref[pl.ds(k * CH, L)]

        if refill:
            nk = k + NBUF

            @pl.when(nk < NCHUNK)
            def _():
                pltpu.async_copy(
                    qa_hbm.at[b, pl.ds(KTC + nk * CH, CH), :],
                    bufs[i], sems[i])

        sd, sl = plsc.sort_key_val(dvec, lab16)
        return _merge_sorted(top_d, top_l, sd, sl)

    def step(s, carry):
        top_d, top_l = carry
        for i in range(NBUF):
            top_d, top_l = consume(s * NBUF + i, i, top_d, top_l, refill=True)
        return top_d, top_l

    top_d = jnp.full((L,), BIG, jnp.float32)
    top_l = jnp.zeros((L,), jnp.int32)
    top_d, top_l = lax.fori_loop(0, STEPS, step, (top_d, top_l))
    for k in range(STEPS * NBUF, NCHUNK):
        top_d, top_l = consume(k, k % NBUF, top_d, top_l, refill=False)

    tdv[...] = top_d
    tlv[...] = top_l
    pltpu.sync_copy(tdv, outd_hbm.at[b])
    pltpu.sync_copy(tlv, outl_hbm.at[b])


def _tc_body(l_ref, qa_ref, o_ref):
    d = qa_ref[...] - l_ref[...][:, None, :]
    o_ref[...] = jnp.sum(d * d, axis=-1)


_tc_dists = pl.pallas_call(
    _tc_body,
    grid=(B // BB, KTC // BK),
    in_specs=[
        pl.BlockSpec((BB, DIM), lambda i, k: (i, 0)),
        pl.BlockSpec((BB, BK, DIM), lambda i, k: (i, k, 0)),
    ],
    out_specs=pl.BlockSpec((BB, BK), lambda i, k: (i, k)),
    out_shape=jax.ShapeDtypeStruct((B, KTC), jnp.float32),
)


def _tc_finish_body(tcd_ref, lab_ref, scd_ref, scl_ref, o_ref):
    KA = KTC + L
    d_all = jnp.concatenate([tcd_ref[...], scd_ref[...]], axis=1)
    lab_all = jnp.concatenate([lab_ref[...][:, :KTC], scl_ref[...]], axis=1)
    kio = jax.lax.broadcasted_iota(jnp.int32, (B, KA), 1)

    cur = d_all
    vals = []
    labs = []
    for _ in range(KNN):
        v = jnp.min(cur, axis=1, keepdims=True)
        cand = jnp.where(cur == v, kio, KA)
        idx = jnp.min(cand, axis=1, keepdims=True)
        hit = kio == idx
        labs.append(jnp.sum(jnp.where(hit, lab_all, 0), axis=1, keepdims=True))
        vals.append(v)
        cur = jnp.where(hit, BIG, cur)

    s = -INV_T * jnp.concatenate(vals, axis=1)
    m = jnp.max(s, axis=1, keepdims=True)
    e = jnp.exp(s - m)
    w = e / jnp.sum(e, axis=1, keepdims=True)

    cio = jax.lax.broadcasted_iota(jnp.int32, (B, N_CLASS), 1)
    acc = jnp.zeros((B, N_CLASS), jnp.float32)
    for r in range(KNN):
        acc = acc + w[:, r:r + 1] * (cio == labs[r]).astype(jnp.float32)
    o_ref[...] = acc


_tc_finish = pl.pallas_call(
    _tc_finish_body,
    out_shape=jax.ShapeDtypeStruct((B, N_CLASS), jnp.float32),
)


def kernel(logits, queue_anchor, queue_label):
    scd, scl = _sc_partial(logits, queue_anchor, queue_label)
    tcd = _tc_dists(logits, queue_anchor)
    return _tc_finish(tcd, queue_label, scd, scl)


# speedup vs baseline: 1.0853x; 1.0026x over previous
import functools

import jax
import jax.numpy as jnp
from jax import lax
from jax.experimental import pallas as pl
from jax.experimental.pallas import tpu as pltpu
from jax.experimental.pallas import tpu_sc as plsc

B = 32
K = 1024
DIM = 2048
KNN = 8
N_CLASS = 16
INV_T = 20.0

NC = 2
NS = 16
L = 16

KTC = 896
KSC = K - KTC

CH = 16
NBUF = 2
NCHUNK = KSC // CH
STEPS = NCHUNK // NBUF
UNROLL = 2
DGRP = DIM // (L * UNROLL)

BK = 128
BB = 8

BIG = 3.0e38

_mesh = plsc.VectorSubcoreMesh(core_axis_name="c", subcore_axis_name="s")
_sc_params = pltpu.CompilerParams(needs_layout_passes=False)


def _merge_sorted(top_d, top_l, sd, sl):
    sdr = jnp.flip(sd)
    slr = jnp.flip(sl)
    sel = top_d <= sdr
    md = jnp.where(sel, top_d, sdr)
    ml = jnp.where(sel, top_l, slr)
    rd, rl = plsc.sort_key_val(md, ml)
    return rd, rl


@functools.partial(
    pl.kernel,
    out_type=(
        jax.ShapeDtypeStruct((B, L), jnp.float32),
        jax.ShapeDtypeStruct((B, L), jnp.int32),
    ),
    mesh=_mesh,
    compiler_params=_sc_params,
    scratch_types=[
        pltpu.VMEM((DIM,), jnp.float32),
        pltpu.VMEM((KSC,), jnp.int32),
        pltpu.VMEM((CH, DIM), jnp.float32),
        pltpu.VMEM((CH, DIM), jnp.float32),
        pltpu.VMEM((L,), jnp.float32),
        pltpu.VMEM((L,), jnp.int32),
        pltpu.SemaphoreType.DMA,
        pltpu.SemaphoreType.DMA,
    ],
)
def _sc_partial(logits_hbm, qa_hbm, ql_hbm, outd_hbm, outl_hbm,
                l_ref, lab_ref, buf0, buf1, tdv, tlv,
                sem0, sem1):
    b = lax.axis_index("s") * NC + lax.axis_index("c")
    bufs = (buf0, buf1)
    sems = (sem0, sem1)

    pltpu.sync_copy(logits_hbm.at[b], l_ref)
    pltpu.sync_copy(ql_hbm.at[b, pl.ds(KTC, KSC)], lab_ref)

    for i in range(NBUF):
        pltpu.async_copy(
            qa_hbm.at[b, pl.ds(KTC + i * CH, CH), :], bufs[i], sems[i])

    def chunk_dists(buf):
        def dim_body(j, accs):
            accs = list(accs)
            for u in range(UNROLL):
                base = (j * UNROLL + u) * L
                lvec = l_ref[pl.ds(base, L)]
                for a in range(CH):
                    d = buf[a, pl.ds(base, L)] - lvec
                    accs[a] = accs[a] + d * d
            return tuple(accs)

        z = jnp.zeros((L,), jnp.float32)
        accs = lax.fori_loop(0, DGRP, dim_body, (z,) * CH)
        lanes = lax.iota(jnp.int32, L)
        dvec = jnp.zeros((L,), jnp.float32)
        for a in range(CH):
            dvec = jnp.where(lanes == a, jnp.sum(accs[a]), dvec)
        return dvec

    def consume(k, i, top_d, top_l, refill):
        src = qa_hbm.at[b, pl.ds(KTC + k * CH, CH), :]
        pltpu.make_async_copy(src, bufs[i], sems[i]).wait()

        dvec = chunk_dists(bufs[i])
        lab16 = lab_ref[pl.ds(k * CH, L)]

        if refill:
            nk = k + NBUF

            @pl.when(nk < NCHUNK)
            def _():
                pltpu.async_copy(
                    qa_hbm.at[b, pl.ds(KTC + nk * CH, CH), :],
                    bufs[i], sems[i])

        sd, sl = plsc.sort_key_val(dvec, lab16)
        return _merge_sorted(top_d, top_l, sd, sl)

    def step(s, carry):
        top_d, top_l = carry
        for i in range(NBUF):
            top_d, top_l = consume(s * NBUF + i, i, top_d, top_l, refill=True)
        return top_d, top_l

    top_d = jnp.full((L,), BIG, jnp.float32)
    top_l = jnp.zeros((L,), jnp.int32)
    top_d, top_l = lax.fori_loop(0, STEPS, step, (top_d, top_l))
    for k in range(STEPS * NBUF, NCHUNK):
        top_d, top_l = consume(k, k % NBUF, top_d, top_l, refill=False)

    tdv[...] = top_d
    tlv[...] = top_l
    pltpu.sync_copy(tdv, outd_hbm.at[b])
    pltpu.sync_copy(tlv, outl_hbm.at[b])


def _tc_body(l_ref, qa_ref, o_ref):
    d = qa_ref[...] - l_ref[...][:, None, :]
    o_ref[...] = jnp.sum(d * d, axis=-1)


_tc_dists = pl.pallas_call(
    _tc_body,
    grid=(B // BB, KTC // BK),
    in_specs=[
        pl.BlockSpec((BB, DIM), lambda i, k: (i, 0)),
        pl.BlockSpec((BB, BK, DIM), lambda i, k: (i, k, 0)),
    ],
    out_specs=pl.BlockSpec((BB, BK), lambda i, k: (i, k)),
    out_shape=jax.ShapeDtypeStruct((B, KTC), jnp.float32),
)


def _tc_finish_body(tcd_ref, lab_ref, scd_ref, scl_ref, o_ref):
    KA = KTC + L
    d_all = jnp.concatenate([tcd_ref[...], scd_ref[...]], axis=1)
    lab_all = jnp.concatenate([lab_ref[...][:, :KTC], scl_ref[...]], axis=1)
    kio = jax.lax.broadcasted_iota(jnp.int32, (B, KA), 1)

    cur = d_all
    vals = []
    labs = []
    for _ in range(KNN):
        v = jnp.min(cur, axis=1, keepdims=True)
        cand = jnp.where(cur == v, kio, KA)
        idx = jnp.min(cand, axis=1, keepdims=True)
        hit = kio == idx
        labs.append(jnp.sum(jnp.where(hit, lab_all, 0), axis=1, keepdims=True))
        vals.append(v)
        cur = jnp.where(hit, BIG, cur)

    s = -INV_T * jnp.concatenate(vals, axis=1)
    m = jnp.max(s, axis=1, keepdims=True)
    e = jnp.exp(s - m)
    w = e / jnp.sum(e, axis=1, keepdims=True)

    cio = jax.lax.broadcasted_iota(jnp.int32, (B, N_CLASS), 1)
    acc = jnp.zeros((B, N_CLASS), jnp.float32)
    for r in range(KNN):
        acc = acc + w[:, r:r + 1] * (cio == labs[r]).astype(jnp.float32)
    o_ref[...] = acc


_tc_finish = pl.pallas_call(
    _tc_finish_body,
    out_shape=jax.ShapeDtypeStruct((B, N_CLASS), jnp.float32),
)


def kernel(logits, queue_anchor, queue_label):
    scd, scl = _sc_partial(logits, queue_anchor, queue_label)
    tcd = _tc_dists(logits, queue_anchor)
    return _tc_finish(tcd, queue_label, scd, scl)
